# baseline (speedup 1.0000x reference)
import functools

import jax
import jax.numpy as jnp
from jax import lax
from jax.experimental import pallas as pl
from jax.experimental.pallas import tpu as pltpu
from jax.experimental.pallas import tpu_sc as plsc

N = 10000
E = 320000
D = 128

NC = 2
NS = 16
NW = NC * NS

NPAD = 10240
CHUNK = 128
NCHUNK = 80
E_W = NCHUNK * CHUNK
EPAD = NW * E_W

ROWS_PER_TILE = NPAD // NS
ROWS_PER_WORKER = NPAD // NW

_mesh = lambda: plsc.VectorSubcoreMesh(
    core_axis_name="c", subcore_axis_name="s", num_cores=NC, num_subcores=NS)
_SC_PARAMS = pltpu.CompilerParams(needs_layout_passes=False)


def _rsqrt16(x):
  i = plsc.bitcast(x, jnp.int32)
  i = 0x5F3759DF - lax.shift_right_arithmetic(i, 1)
  y = plsc.bitcast(i, jnp.float32)
  for _ in range(4):
    y = y * (1.5 - 0.5 * x * y * y)
  return y


@functools.partial(
    pl.kernel,
    out_type=(
        jax.ShapeDtypeStruct((NPAD, D), jnp.float32),
        jax.ShapeDtypeStruct((NC, NS, NPAD), jnp.float32),
    ),
    mesh=_mesh(),
    scratch_types=[
        pltpu.VMEM((NPAD,), jnp.float32),
        pltpu.VMEM((E_W,), jnp.int32),
        pltpu.VMEM((E_W,), jnp.float32),
        pltpu.VMEM((NS, ROWS_PER_TILE), jnp.float32),
        pltpu.VMEM((ROWS_PER_TILE,), jnp.float32),
        pltpu.VMEM((ROWS_PER_TILE, D), jnp.float32),
    ],
    compiler_params=_SC_PARAMS,
)
def _prep(dst_hbm, w_hbm, dinvb_hbm, degp_hbm, deg_v, dst_v, w_v, slab_v,
          dinv_v, stage_v):
  cid = lax.axis_index("c")
  sid = lax.axis_index("s")

  @pl.loop(0, NPAD // 16)
  def _zero(i):
    deg_v[pl.ds(i * 16, 16)] = jnp.zeros((16,), jnp.float32)

  for half in range(2):
    row = sid * 2 + half
    pltpu.sync_copy(dst_hbm.at[row], dst_v)
    pltpu.sync_copy(w_hbm.at[row], w_v)

    @pl.loop(0, E_W // 16, unroll=8)
    def _acc(j):
      idx = dst_v[pl.ds(j * 16, 16)]
      val = w_v[pl.ds(j * 16, 16)]
      plsc.addupdate_scatter(deg_v, [idx], val)

  pltpu.sync_copy(deg_v, degp_hbm.at[cid, sid])
  plsc.subcore_barrier()

  base = sid * ROWS_PER_TILE
  pltpu.sync_copy(degp_hbm.at[cid, :, pl.ds(base, ROWS_PER_TILE)], slab_v)
  for v in range(ROWS_PER_TILE // 16):
    acc = slab_v[0, pl.ds(v * 16, 16)]
    for t in range(1, NS):
      acc = acc + slab_v[t, pl.ds(v * 16, 16)]
    deg16 = acc + 1.0
    dinv_v[pl.ds(v * 16, 16)] = _rsqrt16(deg16)

  writes_half = jnp.logical_or(
      jnp.logical_and(cid == 0, sid < NS // 2),
      jnp.logical_and(cid == 1, sid >= NS // 2))

  @pl.when(writes_half)
  def _write():
    @pl.loop(0, ROWS_PER_TILE, unroll=4)
    def _bcast(r):
      wb = plsc.load_gather(dinv_v, [jnp.zeros((16,), jnp.int32) + r])
      for j in range(D // 16):
        stage_v[r, pl.ds(j * 16, 16)] = wb

    pltpu.sync_copy(stage_v, dinvb_hbm.at[pl.ds(base, ROWS_PER_TILE), :])


@functools.partial(
    pl.kernel,
    out_type=jax.ShapeDtypeStruct((NC, NPAD, D), jnp.float32),
    mesh=_mesh(),
    scratch_types=[
        pltpu.VMEM((NCHUNK, CHUNK), jnp.int32),
        pltpu.VMEM((2, CHUNK), jnp.int32),
        pltpu.VMEM((2, CHUNK), jnp.float32),
        pltpu.VMEM((CHUNK, D), jnp.float32),
        pltpu.VMEM((CHUNK, D), jnp.float32),
        pltpu.VMEM_SHARED((NPAD, D), jnp.float32),
        pltpu.SemaphoreType.DMA,
        pltpu.SemaphoreType.DMA,
        pltpu.SemaphoreType.DMA,
        pltpu.SemaphoreType.DMA,
        pltpu.SemaphoreType.DMA,
        pltpu.SemaphoreType.DMA,
    ],
    compiler_params=_SC_PARAMS,
)
def _spmm(src_hbm, dst_hbm, w_hbm, hp_hbm, out_hbm, dst_v, src2_v, w2_v,
          rows0_v, rows1_v, acc_sh, sem_g0, sem_g1, sem_s0, sem_s1,
          sem_i0, sem_i1):
  cid = lax.axis_index("c")
  sid = lax.axis_index("s")
  wid = sid * NC + cid

  @pl.loop(0, CHUNK)
  def _zero(r):
    for j in range(D // 16):
      rows0_v[r, pl.ds(j * 16, 16)] = jnp.zeros((16,), jnp.float32)

  for k in range(ROWS_PER_TILE // CHUNK):
    pltpu.sync_copy(
        rows0_v, acc_sh.at[pl.ds(sid * ROWS_PER_TILE + k * CHUNK, CHUNK), :])

  pltpu.sync_copy(dst_hbm.at[wid], dst_v)
  plsc.subcore_barrier()

  def _idx_fetch(c, buf, sem):
    pltpu.async_copy(src_hbm.at[wid, pl.ds(c * CHUNK, CHUNK)],
                     src2_v.at[buf], sem)
    pltpu.async_copy(w_hbm.at[wid, pl.ds(c * CHUNK, CHUNK)],
                     w2_v.at[buf], sem)

  def _idx_wait(c, buf, sem):
    pltpu.make_async_copy(src_hbm.at[wid, pl.ds(c * CHUNK, CHUNK)],
                          src2_v.at[buf], sem).wait()
    pltpu.make_async_copy(w_hbm.at[wid, pl.ds(c * CHUNK, CHUNK)],
                          w2_v.at[buf], sem).wait()

  def _scale(rows_v, buf):
    @pl.loop(0, CHUNK, unroll=8)
    def _rows(r):
      zero16 = jnp.zeros((16,), jnp.int32)
      wb = plsc.load_gather(w2_v, [zero16 + buf, zero16 + r])
      for j in range(D // 16):
        rows_v[r, pl.ds(j * 16, 16)] = rows_v[r, pl.ds(j * 16, 16)] * wb

  def _gather(buf, rows_v, sem):
    pltpu.async_copy(hp_hbm.at[src2_v.at[buf]], rows_v, sem)

  def _gather_wait(buf, rows_v, sem):
    pltpu.make_async_copy(hp_hbm.at[src2_v.at[buf]], rows_v, sem).wait()

  def _scatter(c, rows_v, sem):
    pltpu.async_copy(rows_v, acc_sh.at[dst_v.at[c]], sem, add=True)

  def _scatter_wait(c, rows_v, sem):
    pltpu.make_async_copy(rows_v, acc_sh.at[dst_v.at[c]], sem).wait()

  _idx_fetch(0, 0, sem_i0)
  _idx_wait(0, 0, sem_i0)
  _gather(0, rows0_v, sem_g0)
  _idx_fetch(1, 1, sem_i1)

  @pl.loop(0, NCHUNK // 2)
  def _pair(i):
    a = i * 2
    b = a + 1
    _gather_wait(0, rows0_v, sem_g0)
    _scale(rows0_v, 0)

    @pl.when(i > 0)
    def _():
      _scatter_wait(b, rows1_v, sem_s1)

    _idx_wait(b, 1, sem_i1)
    _gather(1, rows1_v, sem_g1)
    _scatter(a, rows0_v, sem_s0)

    @pl.when(i < NCHUNK // 2 - 1)
    def _():
      _idx_fetch(a + 2, 0, sem_i0)

    _gather_wait(1, rows1_v, sem_g1)
    _scale(rows1_v, 1)
    _scatter_wait(a, rows0_v, sem_s0)

    @pl.when(i < NCHUNK // 2 - 1)
    def _():
      _idx_wait(a + 2, 0, sem_i0)
      _gather(0, rows0_v, sem_g0)
      _idx_fetch(b + 2, 1, sem_i1)

    _scatter(b, rows1_v, sem_s1)

  _scatter_wait(NCHUNK - 1, rows1_v, sem_s1)

  plsc.subcore_barrier()
  for k in range(ROWS_PER_TILE // CHUNK):
    rows = pl.ds(sid * ROWS_PER_TILE + k * CHUNK, CHUNK)
    pltpu.sync_copy(acc_sh.at[rows, :], out_hbm.at[cid, rows, :])


BLK = 1024
_GRID = NPAD // BLK


def _tc_first_body(dinvb_ref, x_ref, w_ref, o_ref):
  o_ref[...] = dinvb_ref[...] * jnp.dot(
      x_ref[...], w_ref[...], preferred_element_type=jnp.float32)


def _tc_first(dinvb, xpad, W1):
  return pl.pallas_call(
      _tc_first_body,
      grid=(_GRID,),
      in_specs=[
          pl.BlockSpec((BLK, D), lambda i: (i, 0)),
          pl.BlockSpec((BLK, D), lambda i: (i, 0)),
          pl.BlockSpec((D, D), lambda i: (0, 0)),
      ],
      out_specs=pl.BlockSpec((BLK, D), lambda i: (i, 0)),
      out_shape=jax.ShapeDtypeStruct((NPAD, D), jnp.float32),
  )(dinvb, xpad, W1)


def _tc_mid_body(acc_ref, hp_ref, dinvb_ref, b_ref, w_ref, o_ref):
  h = dinvb_ref[...] * (acc_ref[0] + acc_ref[1] + hp_ref[...]) + b_ref[...]
  h = jnp.maximum(h, 0.0)
  o_ref[...] = dinvb_ref[...] * jnp.dot(
      h, w_ref[...], preferred_element_type=jnp.float32)


def _tc_mid(acc, hp1, dinvb, b1, W2):
  return pl.pallas_call(
      _tc_mid_body,
      grid=(_GRID,),
      in_specs=[
          pl.BlockSpec((NC, BLK, D), lambda i: (0, i, 0)),
          pl.BlockSpec((BLK, D), lambda i: (i, 0)),
          pl.BlockSpec((BLK, D), lambda i: (i, 0)),
          pl.BlockSpec((1, D), lambda i: (0, 0)),
          pl.BlockSpec((D, D), lambda i: (0, 0)),
      ],
      out_specs=pl.BlockSpec((BLK, D), lambda i: (i, 0)),
      out_shape=jax.ShapeDtypeStruct((NPAD, D), jnp.float32),
  )(acc, hp1, dinvb, b1, W2)


def _tc_final_body(acc_ref, hp_ref, dinvb_ref, b_ref, o_ref):
  o_ref[...] = (dinvb_ref[...] * (acc_ref[0] + acc_ref[1] + hp_ref[...])
                + b_ref[...])


def _tc_final(acc, hp2, dinvb, b2):
  return pl.pallas_call(
      _tc_final_body,
      grid=(_GRID,),
      in_specs=[
          pl.BlockSpec((NC, BLK, D), lambda i: (0, i, 0)),
          pl.BlockSpec((BLK, D), lambda i: (i, 0)),
          pl.BlockSpec((BLK, D), lambda i: (i, 0)),
          pl.BlockSpec((1, D), lambda i: (0, 0)),
      ],
      out_specs=pl.BlockSpec((BLK, D), lambda i: (i, 0)),
      out_shape=jax.ShapeDtypeStruct((NPAD, D), jnp.float32),
  )(acc, hp2, dinvb, b2)


def kernel(x, edge_index, edge_weight, W1, b1, W2, b2):
  src = edge_index[0].astype(jnp.int32)
  dst = edge_index[1].astype(jnp.int32)

  srcp = jnp.pad(src, (0, EPAD - E), constant_values=N)
  dstp = jnp.pad(dst, (0, EPAD - E), constant_values=NPAD - 1)
  wp = jnp.pad(edge_weight, (0, EPAD - E), constant_values=0.0)

  src2 = srcp.reshape(NW, E_W)
  dst3 = dstp.reshape(NW, NCHUNK, CHUNK)
  dst2 = dstp.reshape(NW, E_W)
  w2 = wp.reshape(NW, E_W)

  xpad = jnp.pad(x, ((0, NPAD - N), (0, 0)))
  b1r = b1.reshape(1, D)
  b2r = b2.reshape(1, D)

  dinvb, _unused_degp = _prep(dst2, w2)
  hp1 = _tc_first(dinvb, xpad, W1)
  acc1 = _spmm(src2, dst3, w2, hp1)
  hp2 = _tc_mid(acc1, hp1, dinvb, b1r, W2)
  acc2 = _spmm(src2, dst3, w2, hp2)
  out = _tc_final(acc2, hp2, dinvb, b2r)
  return out[:N]


# speedup vs baseline: 1.3923x; 1.3923x over previous
import functools

import jax
import jax.numpy as jnp
from jax import lax
from jax.experimental import pallas as pl
from jax.experimental.pallas import tpu as pltpu
from jax.experimental.pallas import tpu_sc as plsc

N = 10000
E = 320000
D = 128

NC = 2
NS = 16
NW = NC * NS

NPAD = 10240
HALF = NPAD // 2
CHUNK = 128
NCHUNK = 80
E_W = NCHUNK * CHUNK
EPAD = NW * E_W
BCH = NCHUNK + 1
B_CAP = BCH * CHUNK

ROWS_PER_TILE = NPAD // NS
STAGE_PER_TILE = HALF // NS

_mesh = lambda: plsc.VectorSubcoreMesh(
    core_axis_name="c", subcore_axis_name="s", num_cores=NC, num_subcores=NS)
_SC_PARAMS = pltpu.CompilerParams(needs_layout_passes=False)


def _rsqrt16(x):
  i = plsc.bitcast(x, jnp.int32)
  i = 0x5F3759DF - lax.shift_right_arithmetic(i, 1)
  y = plsc.bitcast(i, jnp.float32)
  for _ in range(4):
    y = y * (1.5 - 0.5 * x * y * y)
  return y


@functools.partial(
    pl.kernel,
    out_type=(
        jax.ShapeDtypeStruct((NPAD, D), jnp.float32),
        jax.ShapeDtypeStruct((NC, NS, NPAD), jnp.float32),
    ),
    mesh=_mesh(),
    scratch_types=[
        pltpu.VMEM((NPAD,), jnp.float32),
        pltpu.VMEM((E_W,), jnp.int32),
        pltpu.VMEM((E_W,), jnp.float32),
        pltpu.VMEM((NS, ROWS_PER_TILE), jnp.float32),
        pltpu.VMEM((ROWS_PER_TILE,), jnp.float32),
        pltpu.VMEM((ROWS_PER_TILE, D), jnp.float32),
    ],
    compiler_params=_SC_PARAMS,
)
def _prep(dst_hbm, w_hbm, dinvb_hbm, degp_hbm, deg_v, dst_v, w_v, slab_v,
          dinv_v, stage_v):
  cid = lax.axis_index("c")
  sid = lax.axis_index("s")

  @pl.loop(0, NPAD // 16)
  def _zero(i):
    deg_v[pl.ds(i * 16, 16)] = jnp.zeros((16,), jnp.float32)

  for half in range(2):
    row = sid * 2 + half
    pltpu.sync_copy(dst_hbm.at[row], dst_v)
    pltpu.sync_copy(w_hbm.at[row], w_v)

    @pl.loop(0, E_W // 16, unroll=8)
    def _acc(j):
      idx = dst_v[pl.ds(j * 16, 16)]
      val = w_v[pl.ds(j * 16, 16)]
      plsc.addupdate_scatter(deg_v, [idx], val)

  pltpu.sync_copy(deg_v, degp_hbm.at[cid, sid])
  plsc.subcore_barrier()

  base = sid * ROWS_PER_TILE
  pltpu.sync_copy(degp_hbm.at[cid, :, pl.ds(base, ROWS_PER_TILE)], slab_v)
  for v in range(ROWS_PER_TILE // 16):
    acc = slab_v[0, pl.ds(v * 16, 16)]
    for t in range(1, NS):
      acc = acc + slab_v[t, pl.ds(v * 16, 16)]
    deg16 = acc + 1.0
    dinv_v[pl.ds(v * 16, 16)] = _rsqrt16(deg16)

  writes_half = jnp.logical_or(
      jnp.logical_and(cid == 0, sid < NS // 2),
      jnp.logical_and(cid == 1, sid >= NS // 2))

  @pl.when(writes_half)
  def _write():
    @pl.loop(0, ROWS_PER_TILE, unroll=4)
    def _bcast(r):
      wb = plsc.load_gather(dinv_v, [jnp.zeros((16,), jnp.int32) + r])
      for j in range(D // 16):
        stage_v[r, pl.ds(j * 16, 16)] = wb

    pltpu.sync_copy(stage_v, dinvb_hbm.at[pl.ds(base, ROWS_PER_TILE), :])


@functools.partial(
    pl.kernel,
    out_type=(
        jax.ShapeDtypeStruct((NW, 2, 2, B_CAP), jnp.int32),
        jax.ShapeDtypeStruct((NW, 2, 2, B_CAP), jnp.int32),
        jax.ShapeDtypeStruct((NW, 2, 2, B_CAP), jnp.float32),
        jax.ShapeDtypeStruct((NW, 16), jnp.int32),
    ),
    mesh=_mesh(),
    scratch_types=[
        pltpu.VMEM((E_W,), jnp.int32),
        pltpu.VMEM((E_W,), jnp.int32),
        pltpu.VMEM((E_W,), jnp.float32),
        pltpu.VMEM((B_CAP,), jnp.int32),
        pltpu.VMEM((B_CAP,), jnp.int32),
        pltpu.VMEM((B_CAP,), jnp.float32),
        pltpu.VMEM((B_CAP,), jnp.int32),
        pltpu.VMEM((B_CAP,), jnp.int32),
        pltpu.VMEM((B_CAP,), jnp.float32),
        pltpu.VMEM((16,), jnp.int32),
    ],
    compiler_params=_SC_PARAMS,
)
def _route(src_hbm, dst_hbm, w_hbm, srcr_hbm, dstr_hbm, wr_hbm, cnt_hbm,
           src_v, dst_v, w_v, b0s, b0d, b0w, b1s, b1d, b1w, cnt_v):
  cid = lax.axis_index("c")
  sid = lax.axis_index("s")
  wid = cid * NS + sid
  lane = lax.iota(jnp.int32, 16)
  zero16 = jnp.zeros((16,), jnp.int32)

  pltpu.sync_copy(src_hbm.at[wid], src_v)
  pltpu.sync_copy(dst_hbm.at[wid], dst_v)
  pltpu.sync_copy(w_hbm.at[wid], w_v)

  for p in range(2):
    @pl.loop(0, B_CAP // 16)
    def _zb(i):
      sl = pl.ds(i * 16, 16)
      z = jnp.zeros((16,), jnp.int32)
      zf = jnp.zeros((16,), jnp.float32)
      b0s[sl] = z
      b0d[sl] = z
      b0w[sl] = zf
      b1s[sl] = z
      b1d[sl] = z
      b1w[sl] = zf

    def body(j, carry):
      o0, o1 = carry
      sl = pl.ds(j * 16, 16)
      s16 = src_v[sl]
      d16 = dst_v[sl]
      w16 = w_v[sl]
      if p == 0:
        m = s16 < HALF
        s_loc = s16
      else:
        m = s16 >= HALF
        s_loc = s16 - HALF
      m0 = jnp.logical_and(m, d16 < HALF)
      m1 = jnp.logical_and(m, d16 >= HALF)
      plsc.store_compressed(b0s.at[pl.ds(o0, 16)], s_loc, mask=m0)
      plsc.store_compressed(b0d.at[pl.ds(o0, 16)], d16, mask=m0)
      plsc.store_compressed(b0w.at[pl.ds(o0, 16)], w16, mask=m0)
      plsc.store_compressed(b1s.at[pl.ds(o1, 16)], s_loc, mask=m1)
      plsc.store_compressed(b1d.at[pl.ds(o1, 16)], d16 - HALF, mask=m1)
      plsc.store_compressed(b1w.at[pl.ds(o1, 16)], w16, mask=m1)
      return (o0 + jnp.sum(m0.astype(jnp.int32)),
              o1 + jnp.sum(m1.astype(jnp.int32)))

    o0, o1 = lax.fori_loop(0, E_W // 16, body,
                           (jnp.int32(0), jnp.int32(0)))
    nch0 = lax.shift_right_logical(o0 + CHUNK - 1, 7)
    nch1 = lax.shift_right_logical(o1 + CHUNK - 1, 7)
    mask0 = lane == 0
    plsc.store_scatter(cnt_v, [zero16 + (p * 2)], zero16 + nch0, mask=mask0)
    plsc.store_scatter(cnt_v, [zero16 + (p * 2 + 1)], zero16 + nch1,
                       mask=mask0)
    pltpu.sync_copy(b0s, srcr_hbm.at[wid, p, 0])
    pltpu.sync_copy(b0d, dstr_hbm.at[wid, p, 0])
    pltpu.sync_copy(b0w, wr_hbm.at[wid, p, 0])
    pltpu.sync_copy(b1s, srcr_hbm.at[wid, p, 1])
    pltpu.sync_copy(b1d, dstr_hbm.at[wid, p, 1])
    pltpu.sync_copy(b1w, wr_hbm.at[wid, p, 1])

  pltpu.sync_copy(cnt_v, cnt_hbm.at[wid])


@functools.partial(
    pl.kernel,
    out_type=jax.ShapeDtypeStruct((NPAD, D), jnp.float32),
    mesh=_mesh(),
    scratch_types=[
        pltpu.VMEM((BCH, CHUNK), jnp.int32),
        pltpu.VMEM((B_CAP,), jnp.int32),
        pltpu.VMEM((B_CAP,), jnp.float32),
        pltpu.VMEM((CHUNK, D), jnp.float32),
        pltpu.VMEM((2, 16), jnp.int32),
        pltpu.VMEM_SHARED((HALF, D), jnp.float32),
        pltpu.VMEM_SHARED((HALF, D), jnp.float32),
        pltpu.SemaphoreType.DMA,
    ],
    compiler_params=_SC_PARAMS,
)
def _spmm(srcr_hbm, dstr_hbm, wr_hbm, cnt_hbm, hp_hbm, out_hbm, dst_v, src_v,
          w_v, rows_v, cnt_s, hpst_sh, acc_sh, sem):
  cid = lax.axis_index("c")
  sid = lax.axis_index("s")
  sbase = sid * STAGE_PER_TILE

  @pl.loop(0, CHUNK)
  def _zero(r):
    for j in range(D // 16):
      rows_v[r, pl.ds(j * 16, 16)] = jnp.zeros((16,), jnp.float32)

  pltpu.sync_copy(rows_v, acc_sh.at[pl.ds(sbase, CHUNK), :])
  pltpu.sync_copy(rows_v, acc_sh.at[pl.ds(sbase + CHUNK, CHUNK), :])
  pltpu.sync_copy(rows_v.at[pl.ds(0, 64), :],
                  acc_sh.at[pl.ds(sbase + 2 * CHUNK, 64), :])

  pltpu.sync_copy(cnt_hbm.at[2 * sid], cnt_s.at[0])
  pltpu.sync_copy(cnt_hbm.at[2 * sid + 1], cnt_s.at[1])
  lane = lax.iota(jnp.int32, 16)

  pltpu.sync_copy(hp_hbm.at[pl.ds(sbase, STAGE_PER_TILE), :],
                  hpst_sh.at[pl.ds(sbase, STAGE_PER_TILE), :])
  plsc.subcore_barrier()

  for p in range(2):
    if p == 1:
      plsc.subcore_barrier()
      pltpu.sync_copy(hp_hbm.at[pl.ds(HALF + sbase, STAGE_PER_TILE), :],
                      hpst_sh.at[pl.ds(sbase, STAGE_PER_TILE), :])
      plsc.subcore_barrier()

    for wloc in range(2):
      w_ = 2 * sid + wloc
      cnt16 = cnt_s[wloc, pl.ds(0, 16)]
      nch = jnp.sum(jnp.where(lane == p * 2 + cid, cnt16, 0))
      pltpu.sync_copy(srcr_hbm.at[w_, p, cid], src_v)
      pltpu.sync_copy(dstr_hbm.at[w_, p, cid], dst_v)
      pltpu.sync_copy(wr_hbm.at[w_, p, cid], w_v)

      @pl.loop(0, nch)
      def _chunk(c):
        pltpu.async_copy(hpst_sh.at[src_v.at[pl.ds(c * CHUNK, CHUNK)]],
                         rows_v, sem).wait()

        @pl.loop(0, CHUNK, unroll=8)
        def _scale(r):
          wb = plsc.load_gather(
              w_v, [jnp.zeros((16,), jnp.int32) + c * CHUNK + r])
          for j in range(D // 16):
            rows_v[r, pl.ds(j * 16, 16)] = rows_v[r, pl.ds(j * 16, 16)] * wb

        pltpu.sync_copy(rows_v, acc_sh.at[dst_v.at[c]], add=True)

  plsc.subcore_barrier()
  pltpu.sync_copy(acc_sh.at[pl.ds(sbase, STAGE_PER_TILE), :],
                  out_hbm.at[pl.ds(cid * HALF + sbase, STAGE_PER_TILE), :])


BLK = 1024
_GRID = NPAD // BLK


def _tc_first_body(dinvb_ref, x_ref, w_ref, o_ref):
  o_ref[...] = dinvb_ref[...] * jnp.dot(
      x_ref[...], w_ref[...], preferred_element_type=jnp.float32)


def _tc_first(dinvb, xpad, W1):
  return pl.pallas_call(
      _tc_first_body,
      grid=(_GRID,),
      in_specs=[
          pl.BlockSpec((BLK, D), lambda i: (i, 0)),
          pl.BlockSpec((BLK, D), lambda i: (i, 0)),
          pl.BlockSpec((D, D), lambda i: (0, 0)),
      ],
      out_specs=pl.BlockSpec((BLK, D), lambda i: (i, 0)),
      out_shape=jax.ShapeDtypeStruct((NPAD, D), jnp.float32),
  )(dinvb, xpad, W1)


def _tc_mid_body(acc_ref, hp_ref, dinvb_ref, b_ref, w_ref, o_ref):
  h = dinvb_ref[...] * (acc_ref[...] + hp_ref[...]) + b_ref[...]
  h = jnp.maximum(h, 0.0)
  o_ref[...] = dinvb_ref[...] * jnp.dot(
      h, w_ref[...], preferred_element_type=jnp.float32)


def _tc_mid(acc, hp1, dinvb, b1, W2):
  return pl.pallas_call(
      _tc_mid_body,
      grid=(_GRID,),
      in_specs=[
          pl.BlockSpec((BLK, D), lambda i: (i, 0)),
          pl.BlockSpec((BLK, D), lambda i: (i, 0)),
          pl.BlockSpec((BLK, D), lambda i: (i, 0)),
          pl.BlockSpec((1, D), lambda i: (0, 0)),
          pl.BlockSpec((D, D), lambda i: (0, 0)),
      ],
      out_specs=pl.BlockSpec((BLK, D), lambda i: (i, 0)),
      out_shape=jax.ShapeDtypeStruct((NPAD, D), jnp.float32),
  )(acc, hp1, dinvb, b1, W2)


def _tc_final_body(acc_ref, hp_ref, dinvb_ref, b_ref, o_ref):
  o_ref[...] = (dinvb_ref[...] * (acc_ref[...] + hp_ref[...]) + b_ref[...])


def _tc_final(acc, hp2, dinvb, b2):
  return pl.pallas_call(
      _tc_final_body,
      grid=(_GRID,),
      in_specs=[
          pl.BlockSpec((BLK, D), lambda i: (i, 0)),
          pl.BlockSpec((BLK, D), lambda i: (i, 0)),
          pl.BlockSpec((BLK, D), lambda i: (i, 0)),
          pl.BlockSpec((1, D), lambda i: (0, 0)),
      ],
      out_specs=pl.BlockSpec((BLK, D), lambda i: (i, 0)),
      out_shape=jax.ShapeDtypeStruct((NPAD, D), jnp.float32),
  )(acc, hp2, dinvb, b2)


def kernel(x, edge_index, edge_weight, W1, b1, W2, b2):
  src = edge_index[0].astype(jnp.int32)
  dst = edge_index[1].astype(jnp.int32)

  srcp = jnp.pad(src, (0, EPAD - E), constant_values=N)
  dstp = jnp.pad(dst, (0, EPAD - E), constant_values=NPAD - 1)
  wp = jnp.pad(edge_weight, (0, EPAD - E), constant_values=0.0)

  src2 = srcp.reshape(NW, E_W)
  dst2 = dstp.reshape(NW, E_W)
  w2 = wp.reshape(NW, E_W)

  xpad = jnp.pad(x, ((0, NPAD - N), (0, 0)))
  b1r = b1.reshape(1, D)
  b2r = b2.reshape(1, D)

  dinvb, _unused_degp = _prep(dst2, w2)
  srcr, dstr, wr, cnts = _route(src2, dst2, w2)
  dstr5 = dstr.reshape(NW, 2, 2, BCH, CHUNK)
  hp1 = _tc_first(dinvb, xpad, W1)
  acc1 = _spmm(srcr, dstr5, wr, cnts, hp1)
  hp2 = _tc_mid(acc1, hp1, dinvb, b1r, W2)
  acc2 = _spmm(srcr, dstr5, wr, cnts, hp2)
  out = _tc_final(acc2, hp2, dinvb, b2r)
  return out[:N]


# speedup vs baseline: 1.4678x; 1.0542x over previous
import functools

import jax
import jax.numpy as jnp
from jax import lax
from jax.experimental import pallas as pl
from jax.experimental.pallas import tpu as pltpu
from jax.experimental.pallas import tpu_sc as plsc

N = 10000
E = 320000
D = 128

NC = 2
NS = 16
NW = NC * NS

NPAD = 10240
HALF = NPAD // 2
CHUNK = 128
NCHUNK = 80
E_W = NCHUNK * CHUNK
EPAD = NW * E_W
BCH = NCHUNK + 1
B_CAP = BCH * CHUNK

ROWS_PER_TILE = NPAD // NS
STAGE_PER_TILE = HALF // NS
SPH = 4
QTR = NPAD // SPH
HP_PER_TILE = QTR // NS

_mesh = lambda: plsc.VectorSubcoreMesh(
    core_axis_name="c", subcore_axis_name="s", num_cores=NC, num_subcores=NS)
_SC_PARAMS = pltpu.CompilerParams(needs_layout_passes=False)


def _rsqrt16(x):
  i = plsc.bitcast(x, jnp.int32)
  i = 0x5F3759DF - lax.shift_right_arithmetic(i, 1)
  y = plsc.bitcast(i, jnp.float32)
  for _ in range(4):
    y = y * (1.5 - 0.5 * x * y * y)
  return y


@functools.partial(
    pl.kernel,
    out_type=(
        jax.ShapeDtypeStruct((NPAD, D), jnp.float32),
        jax.ShapeDtypeStruct((NC, NS, NPAD), jnp.float32),
    ),
    mesh=_mesh(),
    scratch_types=[
        pltpu.VMEM((NPAD,), jnp.float32),
        pltpu.VMEM((E_W,), jnp.int32),
        pltpu.VMEM((E_W,), jnp.float32),
        pltpu.VMEM((NS, ROWS_PER_TILE), jnp.float32),
        pltpu.VMEM((ROWS_PER_TILE,), jnp.float32),
        pltpu.VMEM((ROWS_PER_TILE, D), jnp.float32),
    ],
    compiler_params=_SC_PARAMS,
)
def _prep(dst_hbm, w_hbm, dinvb_hbm, degp_hbm, deg_v, dst_v, w_v, slab_v,
          dinv_v, stage_v):
  cid = lax.axis_index("c")
  sid = lax.axis_index("s")

  @pl.loop(0, NPAD // 16)
  def _zero(i):
    deg_v[pl.ds(i * 16, 16)] = jnp.zeros((16,), jnp.float32)

  for half in range(2):
    row = sid * 2 + half
    pltpu.sync_copy(dst_hbm.at[row], dst_v)
    pltpu.sync_copy(w_hbm.at[row], w_v)

    @pl.loop(0, E_W // 16, unroll=8)
    def _acc(j):
      idx = dst_v[pl.ds(j * 16, 16)]
      val = w_v[pl.ds(j * 16, 16)]
      plsc.addupdate_scatter(deg_v, [idx], val)

  pltpu.sync_copy(deg_v, degp_hbm.at[cid, sid])
  plsc.subcore_barrier()

  base = sid * ROWS_PER_TILE
  pltpu.sync_copy(degp_hbm.at[cid, :, pl.ds(base, ROWS_PER_TILE)], slab_v)
  for v in range(ROWS_PER_TILE // 16):
    acc = slab_v[0, pl.ds(v * 16, 16)]
    for t in range(1, NS):
      acc = acc + slab_v[t, pl.ds(v * 16, 16)]
    deg16 = acc + 1.0
    dinv_v[pl.ds(v * 16, 16)] = _rsqrt16(deg16)

  writes_half = jnp.logical_or(
      jnp.logical_and(cid == 0, sid < NS // 2),
      jnp.logical_and(cid == 1, sid >= NS // 2))

  @pl.when(writes_half)
  def _write():
    @pl.loop(0, ROWS_PER_TILE, unroll=4)
    def _bcast(r):
      wb = plsc.load_gather(dinv_v, [jnp.zeros((16,), jnp.int32) + r])
      for j in range(D // 16):
        stage_v[r, pl.ds(j * 16, 16)] = wb

    pltpu.sync_copy(stage_v, dinvb_hbm.at[pl.ds(base, ROWS_PER_TILE), :])


@functools.partial(
    pl.kernel,
    out_type=(
        jax.ShapeDtypeStruct((NW, SPH, 2, B_CAP), jnp.int32),
        jax.ShapeDtypeStruct((NW, SPH, 2, B_CAP), jnp.int32),
        jax.ShapeDtypeStruct((NW, SPH, 2, B_CAP), jnp.float32),
        jax.ShapeDtypeStruct((NW, 16), jnp.int32),
    ),
    mesh=_mesh(),
    scratch_types=[
        pltpu.VMEM((E_W,), jnp.int32),
        pltpu.VMEM((E_W,), jnp.int32),
        pltpu.VMEM((E_W,), jnp.float32),
        pltpu.VMEM((B_CAP,), jnp.int32),
        pltpu.VMEM((B_CAP,), jnp.int32),
        pltpu.VMEM((B_CAP,), jnp.float32),
        pltpu.VMEM((B_CAP,), jnp.int32),
        pltpu.VMEM((B_CAP,), jnp.int32),
        pltpu.VMEM((B_CAP,), jnp.float32),
        pltpu.VMEM((16,), jnp.int32),
    ],
    compiler_params=_SC_PARAMS,
)
def _route(src_hbm, dst_hbm, w_hbm, srcr_hbm, dstr_hbm, wr_hbm, cnt_hbm,
           src_v, dst_v, w_v, b0s, b0d, b0w, b1s, b1d, b1w, cnt_v):
  cid = lax.axis_index("c")
  sid = lax.axis_index("s")
  wid = cid * NS + sid
  lane = lax.iota(jnp.int32, 16)
  zero16 = jnp.zeros((16,), jnp.int32)

  pltpu.sync_copy(src_hbm.at[wid], src_v)
  pltpu.sync_copy(dst_hbm.at[wid], dst_v)
  pltpu.sync_copy(w_hbm.at[wid], w_v)

  for p in range(SPH):
    @pl.loop(0, B_CAP // 16)
    def _zb(i):
      sl = pl.ds(i * 16, 16)
      z = jnp.zeros((16,), jnp.int32)
      zf = jnp.zeros((16,), jnp.float32)
      b0s[sl] = z
      b0d[sl] = z
      b0w[sl] = zf
      b1s[sl] = z
      b1d[sl] = z
      b1w[sl] = zf

    def body(j, carry):
      o0, o1 = carry
      sl = pl.ds(j * 16, 16)
      s16 = src_v[sl]
      d16 = dst_v[sl]
      w16 = w_v[sl]
      lo = p * QTR
      if p == 0:
        m = s16 < QTR
      elif p == SPH - 1:
        m = s16 >= lo
      else:
        m = jnp.logical_and(s16 >= lo, s16 < lo + QTR)
      s_loc = s16 - lo
      m0 = jnp.logical_and(m, d16 < HALF)
      m1 = jnp.logical_and(m, d16 >= HALF)
      plsc.store_compressed(b0s.at[pl.ds(o0, 16)], s_loc, mask=m0)
      plsc.store_compressed(b0d.at[pl.ds(o0, 16)], d16, mask=m0)
      plsc.store_compressed(b0w.at[pl.ds(o0, 16)], w16, mask=m0)
      plsc.store_compressed(b1s.at[pl.ds(o1, 16)], s_loc, mask=m1)
      plsc.store_compressed(b1d.at[pl.ds(o1, 16)], d16 - HALF, mask=m1)
      plsc.store_compressed(b1w.at[pl.ds(o1, 16)], w16, mask=m1)
      return (o0 + jnp.sum(m0.astype(jnp.int32)),
              o1 + jnp.sum(m1.astype(jnp.int32)))

    o0, o1 = lax.fori_loop(0, E_W // 16, body,
                           (jnp.int32(0), jnp.int32(0)))
    nch0 = lax.shift_right_logical(o0 + CHUNK - 1, 7)
    nch1 = lax.shift_right_logical(o1 + CHUNK - 1, 7)
    mask0 = lane == 0
    plsc.store_scatter(cnt_v, [zero16 + (p * 2)], zero16 + nch0, mask=mask0)
    plsc.store_scatter(cnt_v, [zero16 + (p * 2 + 1)], zero16 + nch1,
                       mask=mask0)
    pltpu.sync_copy(b0s, srcr_hbm.at[wid, p, 0])
    pltpu.sync_copy(b0d, dstr_hbm.at[wid, p, 0])
    pltpu.sync_copy(b0w, wr_hbm.at[wid, p, 0])
    pltpu.sync_copy(b1s, srcr_hbm.at[wid, p, 1])
    pltpu.sync_copy(b1d, dstr_hbm.at[wid, p, 1])
    pltpu.sync_copy(b1w, wr_hbm.at[wid, p, 1])

  pltpu.sync_copy(cnt_v, cnt_hbm.at[wid])


@functools.partial(
    pl.kernel,
    out_type=jax.ShapeDtypeStruct((NPAD, D), jnp.float32),
    mesh=_mesh(),
    scratch_types=[
        pltpu.VMEM((BCH, CHUNK), jnp.int32),
        pltpu.VMEM((B_CAP,), jnp.int32),
        pltpu.VMEM((B_CAP,), jnp.float32),
        pltpu.VMEM((CHUNK, D), jnp.float32),
        pltpu.VMEM((CHUNK, D), jnp.float32),
        pltpu.VMEM((2, 16), jnp.int32),
        pltpu.VMEM_SHARED((QTR, D), jnp.float32),
        pltpu.VMEM_SHARED((HALF, D), jnp.float32),
        pltpu.SemaphoreType.DMA,
        pltpu.SemaphoreType.DMA,
        pltpu.SemaphoreType.DMA,
        pltpu.SemaphoreType.DMA,
    ],
    compiler_params=_SC_PARAMS,
)
def _spmm(srcr_hbm, dstr_hbm, wr_hbm, cnt_hbm, hp_hbm, out_hbm, dst_v, src_v,
          w_v, rows_v, rows1_v, cnt_s, hpst_sh, acc_sh, sem_g0, sem_g1,
          sem_s0, sem_s1):
  cid = lax.axis_index("c")
  sid = lax.axis_index("s")
  sbase = sid * STAGE_PER_TILE

  @pl.loop(0, CHUNK)
  def _zero(r):
    for j in range(D // 16):
      rows_v[r, pl.ds(j * 16, 16)] = jnp.zeros((16,), jnp.float32)

  pltpu.sync_copy(rows_v, acc_sh.at[pl.ds(sbase, CHUNK), :])
  pltpu.sync_copy(rows_v, acc_sh.at[pl.ds(sbase + CHUNK, CHUNK), :])
  pltpu.sync_copy(rows_v.at[pl.ds(0, 64), :],
                  acc_sh.at[pl.ds(sbase + 2 * CHUNK, 64), :])

  pltpu.sync_copy(cnt_hbm.at[2 * sid], cnt_s.at[0])
  pltpu.sync_copy(cnt_hbm.at[2 * sid + 1], cnt_s.at[1])
  lane = lax.iota(jnp.int32, 16)

  hbase = sid * HP_PER_TILE
  pltpu.sync_copy(hp_hbm.at[pl.ds(hbase, HP_PER_TILE), :],
                  hpst_sh.at[pl.ds(hbase, HP_PER_TILE), :])
  plsc.subcore_barrier()

  for p in range(SPH):
    if p > 0:
      plsc.subcore_barrier()
      pltpu.sync_copy(hp_hbm.at[pl.ds(p * QTR + hbase, HP_PER_TILE), :],
                      hpst_sh.at[pl.ds(hbase, HP_PER_TILE), :])
      plsc.subcore_barrier()

    for wloc in range(2):
      w_ = 2 * sid + wloc
      cnt16 = cnt_s[wloc, pl.ds(0, 16)]
      nch = jnp.sum(jnp.where(lane == p * 2 + cid, cnt16, 0))
      pltpu.sync_copy(srcr_hbm.at[w_, p, cid], src_v)
      pltpu.sync_copy(dstr_hbm.at[w_, p, cid], dst_v)
      pltpu.sync_copy(wr_hbm.at[w_, p, cid], w_v)

      def _gather(c, rv, sem):
        pltpu.async_copy(hpst_sh.at[src_v.at[pl.ds(c * CHUNK, CHUNK)]],
                         rv, sem)

      def _gather_wait(c, rv, sem):
        pltpu.make_async_copy(hpst_sh.at[src_v.at[pl.ds(c * CHUNK, CHUNK)]],
                              rv, sem).wait()

      def _scale(rv, c):
        @pl.loop(0, CHUNK, unroll=8)
        def _rows(r):
          wb = plsc.load_gather(
              w_v, [jnp.zeros((16,), jnp.int32) + c * CHUNK + r])
          for j in range(D // 16):
            rv[r, pl.ds(j * 16, 16)] = rv[r, pl.ds(j * 16, 16)] * wb

      def _scatter(c, rv, sem):
        pltpu.async_copy(rv, acc_sh.at[dst_v.at[c]], sem, add=True)

      def _scatter_wait(c, rv, sem):
        pltpu.make_async_copy(rv, acc_sh.at[dst_v.at[c]], sem).wait()

      npair = lax.shift_right_logical(nch + 1, 1)

      @pl.when(nch > 0)
      def _():
        _gather(0, rows_v, sem_g0)

      @pl.loop(0, npair)
      def _pair(i):
        a = i * 2
        b = a + 1
        _gather_wait(a, rows_v, sem_g0)
        _scale(rows_v, a)

        @pl.when(i > 0)
        def _():
          _scatter_wait(b, rows1_v, sem_s1)

        @pl.when(b < nch)
        def _():
          _gather(b, rows1_v, sem_g1)

        _scatter(a, rows_v, sem_s0)

        @pl.when(b < nch)
        def _():
          _gather_wait(b, rows1_v, sem_g1)
          _scale(rows1_v, b)

        _scatter_wait(a, rows_v, sem_s0)

        @pl.when(a + 2 < nch)
        def _():
          _gather(a + 2, rows_v, sem_g0)

        @pl.when(b < nch)
        def _():
          _scatter(b, rows1_v, sem_s1)

      @pl.when(jnp.logical_and(nch > 0, (nch & 1) == 0))
      def _():
        _scatter_wait(nch - 1, rows1_v, sem_s1)

  plsc.subcore_barrier()
  pltpu.sync_copy(acc_sh.at[pl.ds(sbase, STAGE_PER_TILE), :],
                  out_hbm.at[pl.ds(cid * HALF + sbase, STAGE_PER_TILE), :])


BLK = 1024
_GRID = NPAD // BLK


def _tc_first_body(dinvb_ref, x_ref, w_ref, o_ref):
  o_ref[...] = dinvb_ref[...] * jnp.dot(
      x_ref[...], w_ref[...], preferred_element_type=jnp.float32)


def _tc_first(dinvb, xpad, W1):
  return pl.pallas_call(
      _tc_first_body,
      grid=(_GRID,),
      in_specs=[
          pl.BlockSpec((BLK, D), lambda i: (i, 0)),
          pl.BlockSpec((BLK, D), lambda i: (i, 0)),
          pl.BlockSpec((D, D), lambda i: (0, 0)),
      ],
      out_specs=pl.BlockSpec((BLK, D), lambda i: (i, 0)),
      out_shape=jax.ShapeDtypeStruct((NPAD, D), jnp.float32),
  )(dinvb, xpad, W1)


def _tc_mid_body(acc_ref, hp_ref, dinvb_ref, b_ref, w_ref, o_ref):
  h = dinvb_ref[...] * (acc_ref[...] + hp_ref[...]) + b_ref[...]
  h = jnp.maximum(h, 0.0)
  o_ref[...] = dinvb_ref[...] * jnp.dot(
      h, w_ref[...], preferred_element_type=jnp.float32)


def _tc_mid(acc, hp1, dinvb, b1, W2):
  return pl.pallas_call(
      _tc_mid_body,
      grid=(_GRID,),
      in_specs=[
          pl.BlockSpec((BLK, D), lambda i: (i, 0)),
          pl.BlockSpec((BLK, D), lambda i: (i, 0)),
          pl.BlockSpec((BLK, D), lambda i: (i, 0)),
          pl.BlockSpec((1, D), lambda i: (0, 0)),
          pl.BlockSpec((D, D), lambda i: (0, 0)),
      ],
      out_specs=pl.BlockSpec((BLK, D), lambda i: (i, 0)),
      out_shape=jax.ShapeDtypeStruct((NPAD, D), jnp.float32),
  )(acc, hp1, dinvb, b1, W2)


def _tc_final_body(acc_ref, hp_ref, dinvb_ref, b_ref, o_ref):
  o_ref[...] = (dinvb_ref[...] * (acc_ref[...] + hp_ref[...]) + b_ref[...])


def _tc_final(acc, hp2, dinvb, b2):
  return pl.pallas_call(
      _tc_final_body,
      grid=(_GRID,),
      in_specs=[
          pl.BlockSpec((BLK, D), lambda i: (i, 0)),
          pl.BlockSpec((BLK, D), lambda i: (i, 0)),
          pl.BlockSpec((BLK, D), lambda i: (i, 0)),
          pl.BlockSpec((1, D), lambda i: (0, 0)),
      ],
      out_specs=pl.BlockSpec((BLK, D), lambda i: (i, 0)),
      out_shape=jax.ShapeDtypeStruct((NPAD, D), jnp.float32),
  )(acc, hp2, dinvb, b2)


def kernel(x, edge_index, edge_weight, W1, b1, W2, b2):
  src = edge_index[0].astype(jnp.int32)
  dst = edge_index[1].astype(jnp.int32)

  srcp = jnp.pad(src, (0, EPAD - E), constant_values=N)
  dstp = jnp.pad(dst, (0, EPAD - E), constant_values=NPAD - 1)
  wp = jnp.pad(edge_weight, (0, EPAD - E), constant_values=0.0)

  src2 = srcp.reshape(NW, E_W)
  dst2 = dstp.reshape(NW, E_W)
  w2 = wp.reshape(NW, E_W)

  xpad = jnp.pad(x, ((0, NPAD - N), (0, 0)))
  b1r = b1.reshape(1, D)
  b2r = b2.reshape(1, D)

  dinvb, _unused_degp = _prep(dst2, w2)
  srcr, dstr, wr, cnts = _route(src2, dst2, w2)
  dstr5 = dstr.reshape(NW, SPH, 2, BCH, CHUNK)
  hp1 = _tc_first(dinvb, xpad, W1)
  acc1 = _spmm(srcr, dstr5, wr, cnts, hp1)
  hp2 = _tc_mid(acc1, hp1, dinvb, b1r, W2)
  acc2 = _spmm(srcr, dstr5, wr, cnts, hp2)
  out = _tc_final(acc2, hp2, dinvb, b2r)
  return out[:N]


# speedup vs baseline: 1.6952x; 1.1549x over previous
import functools

import jax
import jax.numpy as jnp
from jax import lax
from jax.experimental import pallas as pl
from jax.experimental.pallas import tpu as pltpu
from jax.experimental.pallas import tpu_sc as plsc

N = 10000
E = 320000
D = 128

NC = 2
NS = 16
NW = NC * NS

NPAD = 10240
HALF = NPAD // 2
CHUNK = 128
NCHUNK = 80
E_W = NCHUNK * CHUNK
EPAD = NW * E_W
BCH = NCHUNK + 1
B_CAP = BCH * CHUNK

ROWS_PER_TILE = NPAD // NS
STAGE_PER_TILE = HALF // NS
SPH = 4
QTR = NPAD // SPH
HP_PER_TILE = QTR // NS

_mesh = lambda: plsc.VectorSubcoreMesh(
    core_axis_name="c", subcore_axis_name="s", num_cores=NC, num_subcores=NS)
_SC_PARAMS = pltpu.CompilerParams(needs_layout_passes=False)


def _rsqrt16(x):
  i = plsc.bitcast(x, jnp.int32)
  i = 0x5F3759DF - lax.shift_right_arithmetic(i, 1)
  y = plsc.bitcast(i, jnp.float32)
  for _ in range(4):
    y = y * (1.5 - 0.5 * x * y * y)
  return y


@functools.partial(
    pl.kernel,
    out_type=(
        jax.ShapeDtypeStruct((NPAD, D), jnp.float32),
        jax.ShapeDtypeStruct((NC, NS, NPAD), jnp.float32),
    ),
    mesh=_mesh(),
    scratch_types=[
        pltpu.VMEM((NPAD,), jnp.float32),
        pltpu.VMEM((E_W,), jnp.int32),
        pltpu.VMEM((E_W,), jnp.float32),
        pltpu.VMEM((NS, ROWS_PER_TILE), jnp.float32),
        pltpu.VMEM((ROWS_PER_TILE,), jnp.float32),
        pltpu.VMEM((ROWS_PER_TILE, D), jnp.float32),
    ],
    compiler_params=_SC_PARAMS,
)
def _prep(dst_hbm, w_hbm, dinvb_hbm, degp_hbm, deg_v, dst_v, w_v, slab_v,
          dinv_v, stage_v):
  cid = lax.axis_index("c")
  sid = lax.axis_index("s")

  @pl.loop(0, NPAD // 16)
  def _zero(i):
    deg_v[pl.ds(i * 16, 16)] = jnp.zeros((16,), jnp.float32)

  for half in range(2):
    row = sid * 2 + half
    pltpu.sync_copy(dst_hbm.at[row], dst_v)
    pltpu.sync_copy(w_hbm.at[row], w_v)

    @pl.loop(0, E_W // 16, unroll=8)
    def _acc(j):
      idx = dst_v[pl.ds(j * 16, 16)]
      val = w_v[pl.ds(j * 16, 16)]
      plsc.addupdate_scatter(deg_v, [idx], val)

  pltpu.sync_copy(deg_v, degp_hbm.at[cid, sid])
  plsc.subcore_barrier()

  base = sid * ROWS_PER_TILE
  pltpu.sync_copy(degp_hbm.at[cid, :, pl.ds(base, ROWS_PER_TILE)], slab_v)
  for v in range(ROWS_PER_TILE // 16):
    acc = slab_v[0, pl.ds(v * 16, 16)]
    for t in range(1, NS):
      acc = acc + slab_v[t, pl.ds(v * 16, 16)]
    deg16 = acc + 1.0
    dinv_v[pl.ds(v * 16, 16)] = _rsqrt16(deg16)

  writes_half = jnp.logical_or(
      jnp.logical_and(cid == 0, sid < NS // 2),
      jnp.logical_and(cid == 1, sid >= NS // 2))

  @pl.when(writes_half)
  def _write():
    @pl.loop(0, ROWS_PER_TILE, unroll=4)
    def _bcast(r):
      wb = plsc.load_gather(dinv_v, [jnp.zeros((16,), jnp.int32) + r])
      for j in range(D // 16):
        stage_v[r, pl.ds(j * 16, 16)] = wb

    pltpu.sync_copy(stage_v, dinvb_hbm.at[pl.ds(base, ROWS_PER_TILE), :])


@functools.partial(
    pl.kernel,
    out_type=(
        jax.ShapeDtypeStruct((NW, SPH, 2, B_CAP), jnp.int32),
        jax.ShapeDtypeStruct((NW, SPH, 2, B_CAP), jnp.int32),
        jax.ShapeDtypeStruct((NW, SPH, 2, B_CAP), jnp.float32),
        jax.ShapeDtypeStruct((NW, 16), jnp.int32),
    ),
    mesh=_mesh(),
    scratch_types=[
        pltpu.VMEM((E_W,), jnp.int32),
        pltpu.VMEM((E_W,), jnp.int32),
        pltpu.VMEM((E_W,), jnp.float32),
        pltpu.VMEM((B_CAP,), jnp.int32),
        pltpu.VMEM((B_CAP,), jnp.int32),
        pltpu.VMEM((B_CAP,), jnp.float32),
        pltpu.VMEM((B_CAP,), jnp.int32),
        pltpu.VMEM((B_CAP,), jnp.int32),
        pltpu.VMEM((B_CAP,), jnp.float32),
        pltpu.VMEM((16,), jnp.int32),
    ],
    compiler_params=_SC_PARAMS,
)
def _route(src_hbm, dst_hbm, w_hbm, srcr_hbm, dstr_hbm, wr_hbm, cnt_hbm,
           src_v, dst_v, w_v, b0s, b0d, b0w, b1s, b1d, b1w, cnt_v):
  cid = lax.axis_index("c")
  sid = lax.axis_index("s")
  wid = cid * NS + sid
  lane = lax.iota(jnp.int32, 16)
  zero16 = jnp.zeros((16,), jnp.int32)

  pltpu.sync_copy(src_hbm.at[wid], src_v)
  pltpu.sync_copy(dst_hbm.at[wid], dst_v)
  pltpu.sync_copy(w_hbm.at[wid], w_v)

  for p in range(SPH):
    @pl.loop(0, B_CAP // 16)
    def _zb(i):
      sl = pl.ds(i * 16, 16)
      z = jnp.zeros((16,), jnp.int32)
      zf = jnp.zeros((16,), jnp.float32)
      b0s[sl] = z
      b0d[sl] = z
      b0w[sl] = zf
      b1s[sl] = z
      b1d[sl] = z
      b1w[sl] = zf

    def body(j, carry):
      o0, o1 = carry
      sl = pl.ds(j * 16, 16)
      s16 = src_v[sl]
      d16 = dst_v[sl]
      w16 = w_v[sl]
      lo = p * QTR
      if p == 0:
        m = s16 < QTR
      elif p == SPH - 1:
        m = s16 >= lo
      else:
        m = jnp.logical_and(s16 >= lo, s16 < lo + QTR)
      s_loc = s16 - lo
      m0 = jnp.logical_and(m, d16 < HALF)
      m1 = jnp.logical_and(m, d16 >= HALF)
      plsc.store_compressed(b0s.at[pl.ds(o0, 16)], s_loc, mask=m0)
      plsc.store_compressed(b0d.at[pl.ds(o0, 16)], d16, mask=m0)
      plsc.store_compressed(b0w.at[pl.ds(o0, 16)], w16, mask=m0)
      plsc.store_compressed(b1s.at[pl.ds(o1, 16)], s_loc, mask=m1)
      plsc.store_compressed(b1d.at[pl.ds(o1, 16)], d16 - HALF, mask=m1)
      plsc.store_compressed(b1w.at[pl.ds(o1, 16)], w16, mask=m1)
      return (o0 + jnp.sum(m0.astype(jnp.int32)),
              o1 + jnp.sum(m1.astype(jnp.int32)))

    o0, o1 = lax.fori_loop(0, E_W // 16, body,
                           (jnp.int32(0), jnp.int32(0)))
    nch0 = lax.shift_right_logical(o0 + CHUNK - 1, 7)
    nch1 = lax.shift_right_logical(o1 + CHUNK - 1, 7)
    nch0 = ((nch0 + 2) // 3) * 3
    nch1 = ((nch1 + 2) // 3) * 3
    mask0 = lane == 0
    plsc.store_scatter(cnt_v, [zero16 + (p * 2)], zero16 + nch0, mask=mask0)
    plsc.store_scatter(cnt_v, [zero16 + (p * 2 + 1)], zero16 + nch1,
                       mask=mask0)
    pltpu.sync_copy(b0s, srcr_hbm.at[wid, p, 0])
    pltpu.sync_copy(b0d, dstr_hbm.at[wid, p, 0])
    pltpu.sync_copy(b0w, wr_hbm.at[wid, p, 0])
    pltpu.sync_copy(b1s, srcr_hbm.at[wid, p, 1])
    pltpu.sync_copy(b1d, dstr_hbm.at[wid, p, 1])
    pltpu.sync_copy(b1w, wr_hbm.at[wid, p, 1])

  pltpu.sync_copy(cnt_v, cnt_hbm.at[wid])


@functools.partial(
    pl.kernel,
    out_type=jax.ShapeDtypeStruct((NPAD, D), jnp.float32),
    mesh=_mesh(),
    scratch_types=[
        pltpu.VMEM((BCH, CHUNK), jnp.int32),
        pltpu.VMEM((3, CHUNK), jnp.int32),
        pltpu.VMEM((3, CHUNK), jnp.float32),
        pltpu.VMEM((CHUNK, D), jnp.float32),
        pltpu.VMEM((CHUNK, D), jnp.float32),
        pltpu.VMEM((CHUNK, D), jnp.float32),
        pltpu.VMEM((2, 16), jnp.int32),
        pltpu.VMEM_SHARED((QTR, D), jnp.float32),
        pltpu.VMEM_SHARED((HALF, D), jnp.float32),
        pltpu.SemaphoreType.DMA,
        pltpu.SemaphoreType.DMA,
        pltpu.SemaphoreType.DMA,
        pltpu.SemaphoreType.DMA,
        pltpu.SemaphoreType.DMA,
        pltpu.SemaphoreType.DMA,
        pltpu.SemaphoreType.DMA,
        pltpu.SemaphoreType.DMA,
        pltpu.SemaphoreType.DMA,
    ],
    compiler_params=_SC_PARAMS,
)
def _spmm(srcr_hbm, dstr_hbm, wr_hbm, cnt_hbm, hp_hbm, out_hbm, dst_v, src3_v,
          w3_v, rows0_v, rows1_v, rows2_v, cnt_s, hpst_sh, acc_sh, sem_g0,
          sem_g1, sem_g2, sem_s0, sem_s1, sem_s2, sem_i0, sem_i1, sem_i2):
  cid = lax.axis_index("c")
  sid = lax.axis_index("s")
  sbase = sid * STAGE_PER_TILE

  @pl.loop(0, CHUNK)
  def _zero(r):
    for j in range(D // 16):
      rows0_v[r, pl.ds(j * 16, 16)] = jnp.zeros((16,), jnp.float32)

  pltpu.sync_copy(rows0_v, acc_sh.at[pl.ds(sbase, CHUNK), :])
  pltpu.sync_copy(rows0_v, acc_sh.at[pl.ds(sbase + CHUNK, CHUNK), :])
  pltpu.sync_copy(rows0_v.at[pl.ds(0, 64), :],
                  acc_sh.at[pl.ds(sbase + 2 * CHUNK, 64), :])

  pltpu.sync_copy(cnt_hbm.at[2 * sid], cnt_s.at[0])
  pltpu.sync_copy(cnt_hbm.at[2 * sid + 1], cnt_s.at[1])
  lane = lax.iota(jnp.int32, 16)

  hbase = sid * HP_PER_TILE
  pltpu.sync_copy(hp_hbm.at[pl.ds(hbase, HP_PER_TILE), :],
                  hpst_sh.at[pl.ds(hbase, HP_PER_TILE), :])
  plsc.subcore_barrier()

  for p in range(SPH):
    if p > 0:
      plsc.subcore_barrier()
      pltpu.sync_copy(hp_hbm.at[pl.ds(p * QTR + hbase, HP_PER_TILE), :],
                      hpst_sh.at[pl.ds(hbase, HP_PER_TILE), :])
      plsc.subcore_barrier()

    for wloc in range(2):
      w_ = 2 * sid + wloc
      cnt16 = cnt_s[wloc, pl.ds(0, 16)]
      nch = jnp.sum(jnp.where(lane == p * 2 + cid, cnt16, 0))
      pltpu.sync_copy(dstr_hbm.at[w_, p, cid], dst_v)

      rows = (rows0_v, rows1_v, rows2_v)
      sem_g = (sem_g0, sem_g1, sem_g2)
      sem_s = (sem_s0, sem_s1, sem_s2)
      sem_i = (sem_i0, sem_i1, sem_i2)

      def _ifetch(c, k):
        sl = pl.ds(c * CHUNK, CHUNK)
        pltpu.async_copy(srcr_hbm.at[w_, p, cid, sl], src3_v.at[k], sem_i[k])
        pltpu.async_copy(wr_hbm.at[w_, p, cid, sl], w3_v.at[k], sem_i[k])

      def _iwait(c, k):
        sl = pl.ds(c * CHUNK, CHUNK)
        pltpu.make_async_copy(srcr_hbm.at[w_, p, cid, sl], src3_v.at[k],
                              sem_i[k]).wait()
        pltpu.make_async_copy(wr_hbm.at[w_, p, cid, sl], w3_v.at[k],
                              sem_i[k]).wait()

      def _gather(k):
        pltpu.async_copy(hpst_sh.at[src3_v.at[k]], rows[k], sem_g[k])

      def _gwait(k):
        pltpu.make_async_copy(hpst_sh.at[src3_v.at[k]], rows[k],
                              sem_g[k]).wait()

      def _scale(c, k):
        @pl.loop(0, CHUNK, unroll=8)
        def _rows_loop(r):
          z16 = jnp.zeros((16,), jnp.int32)
          wb = plsc.load_gather(w3_v, [z16 + k, z16 + r])
          for j in range(D // 16):
            rows[k][r, pl.ds(j * 16, 16)] = (
                rows[k][r, pl.ds(j * 16, 16)] * wb)

      def _scatter(c, k):
        pltpu.async_copy(rows[k], acc_sh.at[dst_v.at[c]], sem_s[k], add=True)

      def _swait(c, k):
        pltpu.make_async_copy(rows[k], acc_sh.at[dst_v.at[c]],
                              sem_s[k]).wait()

      @pl.when(nch > 0)
      def _p0():
        _ifetch(0, 0)

      @pl.when(nch > 1)
      def _p1():
        _ifetch(1, 1)

      @pl.when(nch > 2)
      def _p2():
        _ifetch(2, 2)

      @pl.when(nch > 0)
      def _p3():
        _iwait(0, 0)
        _gather(0)

      @pl.loop(0, nch // 3)
      def _trip(i):
        for k in range(3):
          c = i * 3 + k
          kp1 = (k + 1) % 3

          @pl.when(c + 1 < nch)
          def _pre(c=c, k=k, kp1=kp1):
            @pl.when(c >= 2)
            def _w(c=c, kp1=kp1):
              _swait(c - 2, kp1)

            _iwait(c + 1, kp1)
            _gather(kp1)

          _gwait(k)
          _scale(c, k)
          _scatter(c, k)

          @pl.when(c + 3 < nch)
          def _nf(c=c, k=k):
            _ifetch(c + 3, k)

      @pl.when(nch > 0)
      def _drain():
        _swait(nch - 3, 0)
        _swait(nch - 2, 1)
        _swait(nch - 1, 2)

  plsc.subcore_barrier()
  pltpu.sync_copy(acc_sh.at[pl.ds(sbase, STAGE_PER_TILE), :],
                  out_hbm.at[pl.ds(cid * HALF + sbase, STAGE_PER_TILE), :])


BLK = 1024
_GRID = NPAD // BLK


def _tc_first_body(dinvb_ref, x_ref, w_ref, o_ref):
  o_ref[...] = dinvb_ref[...] * jnp.dot(
      x_ref[...], w_ref[...], preferred_element_type=jnp.float32)


def _tc_first(dinvb, xpad, W1):
  return pl.pallas_call(
      _tc_first_body,
      grid=(_GRID,),
      in_specs=[
          pl.BlockSpec((BLK, D), lambda i: (i, 0)),
          pl.BlockSpec((BLK, D), lambda i: (i, 0)),
          pl.BlockSpec((D, D), lambda i: (0, 0)),
      ],
      out_specs=pl.BlockSpec((BLK, D), lambda i: (i, 0)),
      out_shape=jax.ShapeDtypeStruct((NPAD, D), jnp.float32),
  )(dinvb, xpad, W1)


def _tc_mid_body(acc_ref, hp_ref, dinvb_ref, b_ref, w_ref, o_ref):
  h = dinvb_ref[...] * (acc_ref[...] + hp_ref[...]) + b_ref[...]
  h = jnp.maximum(h, 0.0)
  o_ref[...] = dinvb_ref[...] * jnp.dot(
      h, w_ref[...], preferred_element_type=jnp.float32)


def _tc_mid(acc, hp1, dinvb, b1, W2):
  return pl.pallas_call(
      _tc_mid_body,
      grid=(_GRID,),
      in_specs=[
          pl.BlockSpec((BLK, D), lambda i: (i, 0)),
          pl.BlockSpec((BLK, D), lambda i: (i, 0)),
          pl.BlockSpec((BLK, D), lambda i: (i, 0)),
          pl.BlockSpec((1, D), lambda i: (0, 0)),
          pl.BlockSpec((D, D), lambda i: (0, 0)),
      ],
      out_specs=pl.BlockSpec((BLK, D), lambda i: (i, 0)),
      out_shape=jax.ShapeDtypeStruct((NPAD, D), jnp.float32),
  )(acc, hp1, dinvb, b1, W2)


def _tc_final_body(acc_ref, hp_ref, dinvb_ref, b_ref, o_ref):
  o_ref[...] = (dinvb_ref[...] * (acc_ref[...] + hp_ref[...]) + b_ref[...])


def _tc_final(acc, hp2, dinvb, b2):
  return pl.pallas_call(
      _tc_final_body,
      grid=(_GRID,),
      in_specs=[
          pl.BlockSpec((BLK, D), lambda i: (i, 0)),
          pl.BlockSpec((BLK, D), lambda i: (i, 0)),
          pl.BlockSpec((BLK, D), lambda i: (i, 0)),
          pl.BlockSpec((1, D), lambda i: (0, 0)),
      ],
      out_specs=pl.BlockSpec((BLK, D), lambda i: (i, 0)),
      out_shape=jax.ShapeDtypeStruct((NPAD, D), jnp.float32),
  )(acc, hp2, dinvb, b2)


def kernel(x, edge_index, edge_weight, W1, b1, W2, b2):
  src = edge_index[0].astype(jnp.int32)
  dst = edge_index[1].astype(jnp.int32)

  srcp = jnp.pad(src, (0, EPAD - E), constant_values=N)
  dstp = jnp.pad(dst, (0, EPAD - E), constant_values=NPAD - 1)
  wp = jnp.pad(edge_weight, (0, EPAD - E), constant_values=0.0)

  src2 = srcp.reshape(NW, E_W)
  dst2 = dstp.reshape(NW, E_W)
  w2 = wp.reshape(NW, E_W)

  xpad = jnp.pad(x, ((0, NPAD - N), (0, 0)))
  b1r = b1.reshape(1, D)
  b2r = b2.reshape(1, D)

  dinvb, _unused_degp = _prep(dst2, w2)
  srcr, dstr, wr, cnts = _route(src2, dst2, w2)
  dstr5 = dstr.reshape(NW, SPH, 2, BCH, CHUNK)
  hp1 = _tc_first(dinvb, xpad, W1)
  acc1 = _spmm(srcr, dstr5, wr, cnts, hp1)
  hp2 = _tc_mid(acc1, hp1, dinvb, b1r, W2)
  acc2 = _spmm(srcr, dstr5, wr, cnts, hp2)
  out = _tc_final(acc2, hp2, dinvb, b2r)
  return out[:N]
